# baseline (speedup 1.0000x reference)
import functools

import jax
import jax.numpy as jnp
from jax import lax
from jax.experimental import pallas as pl
from jax.experimental.pallas import tpu as pltpu
from jax.experimental.pallas import tpu_sc as plsc

_NC = 2
_NS = 16
_NW = _NC * _NS
_L = 16
_SL = 8
_W = 128
_CH = 32
_NB = 8


def _sc_body(rows_per_w, n_j, x_ref, y_ref, o_ref, *refs):
    wid = lax.axis_index("s") * _NC + lax.axis_index("c")
    base = wid * rows_per_w
    n_chunks = rows_per_w // _CH
    y2 = y_ref.reshape(y_ref.shape[0] * y_ref.shape[1], n_j)
    o2 = o_ref.reshape(o_ref.shape[0] * o_ref.shape[1], n_j)
    xvs, yvs, ovs = refs[0:_NB], refs[_NB:2 * _NB], refs[2 * _NB:3 * _NB]
    xss = refs[3 * _NB:4 * _NB]
    yss = refs[4 * _NB:5 * _NB]
    oss = refs[5 * _NB:6 * _NB]

    def start_in(k):
        b = k % _NB
        xsrc = x_ref.at[pl.ds((base + k * _CH) // _SL, _CH // _SL), :, pl.ds(0, _W)]
        ysrc = y2.at[pl.ds(base + k * _CH, _CH), :]
        return (pltpu.async_copy(xsrc, xvs[b], xss[b]),
                pltpu.async_copy(ysrc, yvs[b], yss[b]))

    def start_out(k):
        b = k % _NB
        return pltpu.async_copy(ovs[b], o2.at[pl.ds(base + k * _CH, _CH), :], oss[b])

    iota = lax.iota(jnp.int32, _L)
    copies = {k: start_in(k) for k in range(_NB - 1)}
    stores = {}
    for k in range(n_chunks):
        if k + _NB - 1 < n_chunks:
            copies[k + _NB - 1] = start_in(k + _NB - 1)
        for c in copies.pop(k):
            c.wait()
        if k >= _NB:
            stores.pop(k - _NB).wait()
        xv, yv, ov = xvs[k % _NB], yvs[k % _NB], ovs[k % _NB]
        for t in range(_CH // _L):
            l16 = iota + t * _L
            for j, c in enumerate((3, 1, 2)):
                cj = jnp.full((_L,), j, jnp.int32)
                g = plsc.load_gather(
                    xv,
                    [
                        lax.shift_right_logical(l16, 3),
                        lax.bitwise_and(l16, jnp.int32(7)),
                        jnp.full((_L,), c, jnp.int32),
                    ],
                )
                yj = plsc.load_gather(yv, [l16, cj])
                plsc.store_scatter(ov, [l16, cj], g + yj)
        stores[k] = start_out(k)

    for k in stores:
        stores[k].wait()


def kernel(x, y):
    B, S, D = x.shape
    J = y.shape[-1]
    R = B * S
    rows_per_w = R // _NW

    x3 = x.reshape(R // _SL, _SL, D)

    mesh = plsc.VectorSubcoreMesh(core_axis_name="c", subcore_axis_name="s")
    scratch = (
        [pltpu.VMEM((_CH // _SL, _SL, _W), x.dtype)] * _NB
        + [pltpu.VMEM((_CH, J), x.dtype)] * (2 * _NB)
        + [pltpu.SemaphoreType.DMA] * (3 * _NB)
    )
    out = pl.kernel(
        functools.partial(_sc_body, rows_per_w, J),
        out_type=jax.ShapeDtypeStruct((B, S, J), x.dtype),
        mesh=mesh,
        compiler_params=pltpu.CompilerParams(needs_layout_passes=False),
        scratch_types=scratch,
    )(x3, y)
    return out


# speedup vs baseline: 1.0074x; 1.0074x over previous
import functools

import jax
import jax.numpy as jnp
from jax import lax
from jax.experimental import pallas as pl
from jax.experimental.pallas import tpu as pltpu
from jax.experimental.pallas import tpu_sc as plsc

_NC = 2
_NS = 16
_NW = _NC * _NS
_L = 16
_SL = 8
_W = 128
_CH = 64
_NB = 4


def _sc_body(rows_per_w, n_j, x_ref, y_ref, o_ref, *refs):
    wid = lax.axis_index("c") * _NS + lax.axis_index("s")
    base = wid * rows_per_w
    n_chunks = rows_per_w // _CH
    y2 = y_ref.reshape(y_ref.shape[0] * y_ref.shape[1], n_j)
    o2 = o_ref.reshape(o_ref.shape[0] * o_ref.shape[1], n_j)
    xvs, yvs, ovs = refs[0:_NB], refs[_NB:2 * _NB], refs[2 * _NB:3 * _NB]
    xss = refs[3 * _NB:4 * _NB]
    yss = refs[4 * _NB:5 * _NB]
    oss = refs[5 * _NB:6 * _NB]

    def start_in(k):
        b = k % _NB
        xsrc = x_ref.at[pl.ds((base + k * _CH) // _SL, _CH // _SL), :, pl.ds(0, _W)]
        ysrc = y2.at[pl.ds(base + k * _CH, _CH), :]
        return (pltpu.async_copy(xsrc, xvs[b], xss[b]),
                pltpu.async_copy(ysrc, yvs[b], yss[b]))

    def start_out(k):
        b = k % _NB
        return pltpu.async_copy(ovs[b], o2.at[pl.ds(base + k * _CH, _CH), :], oss[b])

    iota = lax.iota(jnp.int32, _L)
    copies = {k: start_in(k) for k in range(_NB - 1)}
    stores = {}
    for k in range(n_chunks):
        if k + _NB - 1 < n_chunks:
            copies[k + _NB - 1] = start_in(k + _NB - 1)
        for c in copies.pop(k):
            c.wait()
        if k >= _NB:
            stores.pop(k - _NB).wait()
        xv, yv, ov = xvs[k % _NB], yvs[k % _NB], ovs[k % _NB]
        for t in range(_CH // _L):
            l16 = iota + t * _L
            for j, c in enumerate((3, 1, 2)):
                cj = jnp.full((_L,), j, jnp.int32)
                g = plsc.load_gather(
                    xv,
                    [
                        lax.shift_right_logical(l16, 3),
                        lax.bitwise_and(l16, jnp.int32(7)),
                        jnp.full((_L,), c, jnp.int32),
                    ],
                )
                yj = plsc.load_gather(yv, [l16, cj])
                plsc.store_scatter(ov, [l16, cj], g + yj)
        stores[k] = start_out(k)

    for k in stores:
        stores[k].wait()


def kernel(x, y):
    B, S, D = x.shape
    J = y.shape[-1]
    R = B * S
    rows_per_w = R // _NW

    x3 = x.reshape(R // _SL, _SL, D)

    mesh = plsc.VectorSubcoreMesh(core_axis_name="c", subcore_axis_name="s")
    scratch = (
        [pltpu.VMEM((_CH // _SL, _SL, _W), x.dtype)] * _NB
        + [pltpu.VMEM((_CH, J), x.dtype)] * (2 * _NB)
        + [pltpu.SemaphoreType.DMA] * (3 * _NB)
    )
    out = pl.kernel(
        functools.partial(_sc_body, rows_per_w, J),
        out_type=jax.ShapeDtypeStruct((B, S, J), x.dtype),
        mesh=mesh,
        compiler_params=pltpu.CompilerParams(needs_layout_passes=False),
        scratch_types=scratch,
    )(x3, y)
    return out


# speedup vs baseline: 1.0087x; 1.0013x over previous
import functools

import jax
import jax.numpy as jnp
from jax import lax
from jax.experimental import pallas as pl
from jax.experimental.pallas import tpu as pltpu
from jax.experimental.pallas import tpu_sc as plsc

_NC = 2
_NS = 16
_NW = _NC * _NS
_L = 16
_SL = 8
_W = 128
_CH = 64
_NB = 4


def _sc_body(rows_per_w, n_j, x_ref, y_ref, o_ref, *refs):
    wid = lax.axis_index("c") * _NS + lax.axis_index("s")
    base = wid * rows_per_w
    n_chunks = rows_per_w // _CH
    y2 = y_ref.reshape(y_ref.shape[0] * y_ref.shape[1], n_j)
    o2 = o_ref.reshape(o_ref.shape[0] * o_ref.shape[1], n_j)
    xvs, yvs, ovs = refs[0:_NB], refs[_NB:2 * _NB], refs[2 * _NB:3 * _NB]
    xss = refs[3 * _NB:4 * _NB]
    yss = refs[4 * _NB:5 * _NB]
    oss = refs[5 * _NB:6 * _NB]

    def start_in(k):
        b = k % _NB
        xsrc = x_ref.at[pl.ds((base + k * _CH) // _SL, _CH // _SL), :, pl.ds(0, _W)]
        ysrc = y2.at[pl.ds(base + k * _CH, _CH), :]
        return (pltpu.async_copy(xsrc, xvs[b], xss[b]),
                pltpu.async_copy(ysrc, yvs[b], yss[b]))

    def start_out(k):
        b = k % _NB
        return pltpu.async_copy(ovs[b], o2.at[pl.ds(base + k * _CH, _CH), :], oss[b])

    iota = lax.iota(jnp.int32, _L)
    copies = {k: start_in(k) for k in range(_NB - 1)}
    stores = {}
    l16s = [iota + t * _L for t in range(_CH // _L)]
    row_hi = [lax.shift_right_logical(l, 3) for l in l16s]
    row_lo = [lax.bitwise_and(l, jnp.int32(7)) for l in l16s]
    cols = {c: jnp.full((_L,), c, jnp.int32) for c in (0, 1, 2, 3)}

    for k in range(n_chunks):
        if k + _NB - 1 < n_chunks:
            copies[k + _NB - 1] = start_in(k + _NB - 1)
        for c in copies.pop(k):
            c.wait()
        if k >= _NB:
            stores.pop(k - _NB).wait()
        xv, yv, ov = xvs[k % _NB], yvs[k % _NB], ovs[k % _NB]
        for t in range(_CH // _L):
            for j, c in enumerate((3, 1, 2)):
                g = plsc.load_gather(xv, [row_hi[t], row_lo[t], cols[c]])
                yj = plsc.load_gather(yv, [l16s[t], cols[j]])
                plsc.store_scatter(ov, [l16s[t], cols[j]], g + yj)
        stores[k] = start_out(k)

    for k in stores:
        stores[k].wait()


def kernel(x, y):
    B, S, D = x.shape
    J = y.shape[-1]
    R = B * S
    rows_per_w = R // _NW

    x3 = x.reshape(R // _SL, _SL, D)

    mesh = plsc.VectorSubcoreMesh(core_axis_name="c", subcore_axis_name="s")
    scratch = (
        [pltpu.VMEM((_CH // _SL, _SL, _W), x.dtype)] * _NB
        + [pltpu.VMEM((_CH, J), x.dtype)] * (2 * _NB)
        + [pltpu.SemaphoreType.DMA] * (3 * _NB)
    )
    out = pl.kernel(
        functools.partial(_sc_body, rows_per_w, J),
        out_type=jax.ShapeDtypeStruct((B, S, J), x.dtype),
        mesh=mesh,
        compiler_params=pltpu.CompilerParams(needs_layout_passes=False),
        scratch_types=scratch,
    )(x3, y)
    return out


# speedup vs baseline: 1.0094x; 1.0007x over previous
import functools

import jax
import jax.numpy as jnp
from jax import lax
from jax.experimental import pallas as pl
from jax.experimental.pallas import tpu as pltpu
from jax.experimental.pallas import tpu_sc as plsc

_NC = 2
_NS = 16
_NW = _NC * _NS
_L = 16
_SL = 8
_W = 128
_CH = 64
_NB = 4


def _sc_body(rows_per_w, n_j, x_ref, y_ref, o_ref, *refs):
    wid = lax.axis_index("c") * _NS + lax.axis_index("s")
    base = wid * rows_per_w
    n_chunks = rows_per_w // _CH
    y2 = y_ref
    o2 = o_ref
    xvs, yvs, ovs = refs[0:_NB], refs[_NB:2 * _NB], refs[2 * _NB:3 * _NB]
    xss = refs[3 * _NB:4 * _NB]
    yss = refs[4 * _NB:5 * _NB]
    oss = refs[5 * _NB:6 * _NB]

    def start_in(k):
        b = k % _NB
        xsrc = x_ref.at[pl.ds((base + k * _CH) // _SL, _CH // _SL), :, pl.ds(0, _W)]
        ysrc = y2.at[pl.ds((base + k * _CH) // _SL, _CH // _SL), :, :]
        return (pltpu.async_copy(xsrc, xvs[b], xss[b]),
                pltpu.async_copy(ysrc, yvs[b], yss[b]))

    def start_out(k):
        b = k % _NB
        return pltpu.async_copy(
            ovs[b], o2.at[pl.ds((base + k * _CH) // _SL, _CH // _SL), :, :], oss[b])

    iota = lax.iota(jnp.int32, _L)
    copies = {k: start_in(k) for k in range(_NB - 1)}
    stores = {}
    l16s = [iota + t * _L for t in range(_CH // _L)]
    row_hi = [lax.shift_right_logical(l, 3) for l in l16s]
    row_lo = [lax.bitwise_and(l, jnp.int32(7)) for l in l16s]
    cols = {c: jnp.full((_L,), c, jnp.int32) for c in (0, 1, 2, 3)}

    for k in range(n_chunks):
        if k + _NB - 1 < n_chunks:
            copies[k + _NB - 1] = start_in(k + _NB - 1)
        for c in copies.pop(k):
            c.wait()
        if k >= _NB:
            stores.pop(k - _NB).wait()
        xv, yv, ov = xvs[k % _NB], yvs[k % _NB], ovs[k % _NB]
        for t in range(_CH // _L):
            for j, c in enumerate((3, 1, 2)):
                g = plsc.load_gather(xv, [row_hi[t], row_lo[t], cols[c]])
                yj = plsc.load_gather(yv, [row_hi[t], row_lo[t], cols[j]])
                plsc.store_scatter(ov, [row_hi[t], row_lo[t], cols[j]], g + yj)
        stores[k] = start_out(k)

    for k in stores:
        stores[k].wait()


def kernel(x, y):
    B, S, D = x.shape
    J = y.shape[-1]
    R = B * S
    rows_per_w = R // _NW

    x3 = x.reshape(R // _SL, _SL, D)
    y3 = y.reshape(R // _SL, _SL, J)

    mesh = plsc.VectorSubcoreMesh(core_axis_name="c", subcore_axis_name="s")
    scratch = (
        [pltpu.VMEM((_CH // _SL, _SL, _W), x.dtype)] * _NB
        + [pltpu.VMEM((_CH // _SL, _SL, J), x.dtype)] * (2 * _NB)
        + [pltpu.SemaphoreType.DMA] * (3 * _NB)
    )
    out = pl.kernel(
        functools.partial(_sc_body, rows_per_w, J),
        out_type=jax.ShapeDtypeStruct((R // _SL, _SL, J), x.dtype),
        mesh=mesh,
        compiler_params=pltpu.CompilerParams(needs_layout_passes=False),
        scratch_types=scratch,
    )(x3, y3)
    return out.reshape(B, S, J)
